# baseline (speedup 1.0000x reference)
import jax
import jax.numpy as jnp
from jax import lax
from jax.experimental import pallas as pl
from jax.experimental.pallas import tpu as pltpu
from jax.experimental.pallas import tpu_sc as plsc

VOCAB = 1000000
EMBED_DIM = 64
NUM_PROMPT = 3
PID_BASE = VOCAB - NUM_PROMPT

NC, NS, L = 2, 16, 16
NW = NC * NS
SEQ_CHUNK = 8
G0 = 128
G1 = 200 - G0


def _sc_body(idx_hbm, table_hbm, pp_hbm, out_hbm, idx_v, rows_v, pp_v, sem):
    n_seq, seq_len = idx_hbm.shape
    seq_per_w = n_seq // NW
    chunks_per_w = seq_per_w // SEQ_CHUNK

    wid = lax.axis_index("s") * NC + lax.axis_index("c")
    base_seq = wid * seq_per_w

    pltpu.sync_copy(pp_hbm, pp_v)

    iota16 = lax.iota(jnp.int32, L)
    n_grp = (seq_len + L - 1) // L

    def do_chunk(g, _):
        seq0 = pl.multiple_of(base_seq + g * SEQ_CHUNK, SEQ_CHUNK)
        pltpu.sync_copy(idx_hbm.at[pl.ds(seq0, SEQ_CHUNK)], idx_v)
        for s in range(SEQ_CHUNK):
            pltpu.async_copy(
                table_hbm.at[idx_v.at[s].at[pl.ds(0, G0)]],
                rows_v.at[s].at[pl.ds(0, G0)],
                sem,
            )
            pltpu.async_copy(
                table_hbm.at[idx_v.at[s].at[pl.ds(G0, G1)]],
                rows_v.at[s].at[pl.ds(G0, G1)],
                sem,
            )
        for s in range(SEQ_CHUNK):
            pltpu.make_async_copy(
                table_hbm.at[idx_v.at[s].at[pl.ds(0, G0)]],
                rows_v.at[s].at[pl.ds(0, G0)],
                sem,
            ).wait()
            pltpu.make_async_copy(
                table_hbm.at[idx_v.at[s].at[pl.ds(G0, G1)]],
                rows_v.at[s].at[pl.ds(G0, G1)],
                sem,
            ).wait()

        def fix_group(g2, _):
            s = g2 // n_grp
            w = g2 % n_grp
            off = jnp.minimum(w * L, seq_len - L)
            ivec = idx_v[s, pl.ds(off, L)]
            cond = ivec >= PID_BASE
            cnt = jnp.sum(cond.astype(jnp.int32))

            @pl.when(cnt > 0)
            def _():
                def fix_lane(_, mask):
                    lane = jnp.min(jnp.where(mask > 0, iota16, L))
                    k = jnp.max(jnp.where(iota16 == lane, ivec - PID_BASE, -1))
                    row = off + lane
                    row_splat = jnp.broadcast_to(row, (L,)).astype(jnp.int32)
                    s_splat = jnp.broadcast_to(s, (L,)).astype(jnp.int32)
                    for q in range(EMBED_DIM // L):
                        val = pp_v[pl.ds(k * EMBED_DIM + q * L, L)]
                        plsc.store_scatter(
                            rows_v, [s_splat, row_splat, iota16 + q * L], val)
                    return mask & (iota16 != lane).astype(jnp.int32)

                lax.fori_loop(0, cnt, fix_lane, cond.astype(jnp.int32))

            return 0

        lax.fori_loop(0, SEQ_CHUNK * n_grp, fix_group, 0)

        pltpu.sync_copy(rows_v, out_hbm.at[pl.ds(seq0, SEQ_CHUNK)])
        return 0

    lax.fori_loop(0, chunks_per_w, do_chunk, 0)


@jax.jit
def _run(idx, table, pp_flat):
    n_seq, seq_len = idx.shape
    mesh = plsc.VectorSubcoreMesh(core_axis_name="c", subcore_axis_name="s")
    return pl.kernel(
        _sc_body,
        out_type=jax.ShapeDtypeStruct((n_seq, seq_len, EMBED_DIM), jnp.float32),
        mesh=mesh,
        scratch_types=[
            pltpu.VMEM((SEQ_CHUNK, 200), jnp.int32),
            pltpu.VMEM((SEQ_CHUNK, 200, EMBED_DIM), jnp.float32),
            pltpu.VMEM((NUM_PROMPT * EMBED_DIM,), jnp.float32),
            pltpu.SemaphoreType.DMA,
        ],
        compiler_params=pltpu.CompilerParams(
            use_tc_tiling_on_sc=False, needs_layout_passes=False),
    )(idx, table, pp_flat)


def kernel(input, table, prompt_params):
    return _run(input.astype(jnp.int32), table, prompt_params.reshape(-1))


# speedup vs baseline: 1.0130x; 1.0130x over previous
import jax
import jax.numpy as jnp
from jax import lax
from jax.experimental import pallas as pl
from jax.experimental.pallas import tpu as pltpu
from jax.experimental.pallas import tpu_sc as plsc

VOCAB = 1000000
EMBED_DIM = 64
NUM_PROMPT = 3
PID_BASE = VOCAB - NUM_PROMPT

NC, NS, L = 2, 16, 16
NW = NC * NS
SEQ_CHUNK = 8
G0 = 128
G1 = 200 - G0


def _sc_body(idxt_hbm, table_hbm, pp_hbm, out_hbm,
             idxt_v, idx_v, rows_v, pp_v, *sems):
    seq_len, n_seq = idxt_hbm.shape
    seq_per_w = n_seq // NW
    chunks_per_w = seq_per_w // SEQ_CHUNK

    wid = lax.axis_index("s") * NC + lax.axis_index("c")
    base_seq = wid * seq_per_w

    pltpu.sync_copy(pp_hbm, pp_v)

    iota16 = lax.iota(jnp.int32, L)
    n_grp = (seq_len + L - 1) // L
    n_tr = SEQ_CHUNK * seq_len // L

    def do_chunk(g, _):
        seq0 = pl.multiple_of(base_seq + g * SEQ_CHUNK, SEQ_CHUNK)
        pltpu.sync_copy(idxt_hbm.at[:, pl.ds(seq0, SEQ_CHUNK)], idxt_v)

        def tr(j, _):
            f = j * L + iota16
            s_vec = f // seq_len
            p_vec = f - s_vec * seq_len
            vals = plsc.load_gather(idxt_v, [p_vec, s_vec])
            plsc.store_scatter(idx_v, [s_vec, p_vec], vals)
            return 0

        lax.fori_loop(0, n_tr, tr, 0)

        for s in range(SEQ_CHUNK):
            pltpu.async_copy(
                table_hbm.at[idx_v.at[s].at[pl.ds(0, G0)]],
                rows_v.at[s].at[pl.ds(0, G0)],
                sems[s],
            )
            pltpu.async_copy(
                table_hbm.at[idx_v.at[s].at[pl.ds(G0, G1)]],
                rows_v.at[s].at[pl.ds(G0, G1)],
                sems[s],
            )

        def fix_group(g2, _):
            s = g2 // n_grp
            w = g2 % n_grp
            off = jnp.minimum(w * L, seq_len - L)
            ivec = idx_v[s, pl.ds(off, L)]
            cond = ivec >= PID_BASE
            cnt = jnp.sum(cond.astype(jnp.int32))

            @pl.when(cnt > 0)
            def _():
                def fix_lane(_, mask):
                    lane = jnp.min(jnp.where(mask > 0, iota16, L))
                    k = jnp.max(jnp.where(iota16 == lane, ivec - PID_BASE, -1))
                    row = off + lane
                    row_splat = jnp.broadcast_to(row, (L,)).astype(jnp.int32)
                    s_splat = jnp.broadcast_to(s, (L,)).astype(jnp.int32)
                    for q in range(EMBED_DIM // L):
                        val = pp_v[pl.ds(k * EMBED_DIM + q * L, L)]
                        plsc.store_scatter(
                            rows_v, [s_splat, row_splat, iota16 + q * L], val)
                    return mask & (iota16 != lane).astype(jnp.int32)

                lax.fori_loop(0, cnt, fix_lane, cond.astype(jnp.int32))

            return 0

        for s in range(SEQ_CHUNK):
            pltpu.make_async_copy(
                table_hbm.at[idx_v.at[s].at[pl.ds(0, G0)]],
                rows_v.at[s].at[pl.ds(0, G0)],
                sems[s],
            ).wait()
            pltpu.make_async_copy(
                table_hbm.at[idx_v.at[s].at[pl.ds(G0, G1)]],
                rows_v.at[s].at[pl.ds(G0, G1)],
                sems[s],
            ).wait()
            lax.fori_loop(s * n_grp, (s + 1) * n_grp, fix_group, 0)

        pltpu.sync_copy(rows_v, out_hbm.at[pl.ds(seq0, SEQ_CHUNK)])
        return 0

    lax.fori_loop(0, chunks_per_w, do_chunk, 0)


@jax.jit
def _run(idxt, table, pp_flat):
    seq_len, n_seq = idxt.shape
    mesh = plsc.VectorSubcoreMesh(core_axis_name="c", subcore_axis_name="s")
    return pl.kernel(
        _sc_body,
        out_type=jax.ShapeDtypeStruct((n_seq, seq_len, EMBED_DIM), jnp.float32),
        mesh=mesh,
        scratch_types=[
            pltpu.VMEM((200, SEQ_CHUNK), jnp.int32),
            pltpu.VMEM((SEQ_CHUNK, 200), jnp.int32),
            pltpu.VMEM((SEQ_CHUNK, 200, EMBED_DIM), jnp.float32),
            pltpu.VMEM((NUM_PROMPT * EMBED_DIM,), jnp.float32),
        ] + [pltpu.SemaphoreType.DMA] * SEQ_CHUNK,
        compiler_params=pltpu.CompilerParams(
            use_tc_tiling_on_sc=False, needs_layout_passes=False),
    )(idxt, table, pp_flat)


def kernel(input, table, prompt_params):
    return _run(input.astype(jnp.int32).T, table, prompt_params.reshape(-1))
